# baseline (speedup 1.0000x reference)
import functools

import jax
import jax.numpy as jnp
from jax import lax
from jax.experimental import pallas as pl
from jax.experimental.pallas import tpu as pltpu
from jax.experimental.pallas import tpu_sc as plsc

_R = 1024
_L = 16


def _scores(q, db):
    cn = jnp.sum(db * db, axis=0, keepdims=True)
    return cn - 2.0 * lax.dot_general(
        q, db, (((0,), (0,)), ((), ())),
        preferred_element_type=jnp.float32)


def _select(s, colf, out_ref, nsel):
    inf = jnp.float32(jnp.inf)

    def argat(m):
        ii = jnp.min(jnp.where(s == m, colf, inf), axis=1, keepdims=True)
        return ii[:, 0].astype(jnp.int32)

    m1 = jnp.min(s, axis=1, keepdims=True)
    if nsel == 1:
        out_ref[0, 0, :] = argat(m1)
    else:
        m2 = jnp.min(jnp.where(s > m1, s, inf), axis=1, keepdims=True)
        out_ref[0, 0, :] = argat(m2)
        m3 = jnp.min(jnp.where(s > m2, s, inf), axis=1, keepdims=True)
        out_ref[0, 1, :] = argat(m3)


def _knn_self_body(q_ref, db_ref, out_ref):
    s = _scores(q_ref[0], db_ref[0])
    colf = lax.broadcasted_iota(jnp.int32, s.shape, 1).astype(jnp.float32)
    _select(s, colf, out_ref, 2)


def _knn_adv_body(q_ref, dba_ref, dbp_ref, outa_ref, outp_ref):
    sa = _scores(q_ref[0], dba_ref[0])
    colf = lax.broadcasted_iota(jnp.int32, sa.shape, 1).astype(jnp.float32)
    _select(sa, colf, outa_ref, 2)
    _select(_scores(q_ref[0], dbp_ref[0]), colf, outp_ref, 1)


def _knn_self(pc):
    b, _, n = pc.shape
    return pl.pallas_call(
        _knn_self_body,
        grid=(b, n // _R),
        in_specs=[
            pl.BlockSpec((1, 3, _R), lambda bb, r: (bb, 0, r)),
            pl.BlockSpec((1, 3, n), lambda bb, r: (bb, 0, 0)),
        ],
        out_specs=pl.BlockSpec((1, 2, _R), lambda bb, r: (bb, 0, r)),
        out_shape=jax.ShapeDtypeStruct((b, 2, n), jnp.int32),
    )(pc, pc)


def _knn_adv(adv, pc):
    b, _, n = adv.shape
    return pl.pallas_call(
        _knn_adv_body,
        grid=(b, n // _R),
        in_specs=[
            pl.BlockSpec((1, 3, _R), lambda bb, r: (bb, 0, r)),
            pl.BlockSpec((1, 3, n), lambda bb, r: (bb, 0, 0)),
            pl.BlockSpec((1, 3, n), lambda bb, r: (bb, 0, 0)),
        ],
        out_specs=[
            pl.BlockSpec((1, 2, _R), lambda bb, r: (bb, 0, r)),
            pl.BlockSpec((1, 1, _R), lambda bb, r: (bb, 0, r)),
        ],
        out_shape=[
            jax.ShapeDtypeStruct((b, 2, n), jnp.int32),
            jax.ShapeDtypeStruct((b, 1, n), jnp.int32),
        ],
    )(adv, adv, pc)


def _rsqrt(s):
    i = plsc.bitcast(s, jnp.int32)
    i = jnp.int32(0x5F3759DF) - (i >> 1)
    y = plsc.bitcast(i, jnp.float32)
    for _ in range(3):
        y = y * (1.5 - 0.5 * s * y * y)
    return y


def _sc_grid():
    info = plsc.get_sparse_core_info()
    return info.num_cores, info.num_subcores, info.num_cores * info.num_subcores


def _curv_sc(pcx, pcy, pcz, ax, ay, az, nx, ny, nz, io1, io2, i21, i22, i1n):
    b, n = pcx.shape
    nc, ns, nw = _sc_grid()
    ppw = b * n // nw
    per_b = n // ppw
    bpc = ns // per_b
    mesh = plsc.VectorSubcoreMesh(core_axis_name="c", subcore_axis_name="s")

    @functools.partial(
        pl.kernel, mesh=mesh,
        compiler_params=pltpu.CompilerParams(needs_layout_passes=False),
        out_type=jax.ShapeDtypeStruct((nw, _L), jnp.float32),
        scratch_types=[
            pltpu.VMEM((n,), jnp.float32),
            pltpu.VMEM((n,), jnp.float32),
            pltpu.VMEM((n,), jnp.float32),
            pltpu.VMEM((n,), jnp.float32),
            pltpu.VMEM((n,), jnp.float32),
            pltpu.VMEM((n,), jnp.float32),
            pltpu.VMEM((n,), jnp.float32),
            pltpu.VMEM((n,), jnp.float32),
            pltpu.VMEM((n,), jnp.float32),
            pltpu.VMEM((ppw,), jnp.int32),
            pltpu.VMEM((ppw,), jnp.int32),
            pltpu.VMEM((ppw,), jnp.int32),
            pltpu.VMEM((ppw,), jnp.int32),
            pltpu.VMEM((ppw,), jnp.int32),
            pltpu.VMEM((ppw,), jnp.float32),
            pltpu.VMEM((n,), jnp.float32),
            pltpu.VMEM((_L,), jnp.float32),
            pltpu.VMEM_SHARED((2, n), jnp.float32),
        ],
    )
    def k(pcx_h, pcy_h, pcz_h, ax_h, ay_h, az_h, nx_h, ny_h, nz_h,
          io1_h, io2_h, i21_h, i22_h, i1_h, out_h,
          pxv, pyv, pzv, axv, ayv, azv, nxv, nyv, nzv,
          io1v, io2v, i21v, i22v, i1v, ov, kapv, accv, kshared):
        cid = lax.axis_index("c")
        sid = lax.axis_index("s")
        wid = cid * ns + sid
        bb = wid // per_b
        bloc = bb % bpc
        base = (wid % per_b) * ppw
        for src, dst in ((pcx_h, pxv), (pcy_h, pyv), (pcz_h, pzv),
                         (ax_h, axv), (ay_h, ayv), (az_h, azv),
                         (nx_h, nxv), (ny_h, nyv), (nz_h, nzv)):
            pltpu.sync_copy(src.at[bb], dst)
        for src, dst in ((io1_h, io1v), (io2_h, io2v), (i21_h, i21v),
                         (i22_h, i22v), (i1_h, i1v)):
            pltpu.sync_copy(src.at[bb, pl.ds(base, ppw)], dst)

        def kappa_body(i, carry):
            sl = pl.ds(i * _L, _L)
            gsl = pl.ds(base + i * _L, _L)
            sx, sy, sz = pxv[gsl], pyv[gsl], pzv[gsl]
            mx, my, mz = nxv[gsl], nyv[gsl], nzv[gsl]
            acc = jnp.zeros((_L,), jnp.float32)
            for jv in (io1v[sl], io2v[sl]):
                vx = plsc.load_gather(pxv, [jv]) - sx
                vy = plsc.load_gather(pyv, [jv]) - sy
                vz = plsc.load_gather(pzv, [jv]) - sz
                r = _rsqrt(vx * vx + vy * vy + vz * vz)
                acc = acc + jnp.abs(vx * mx + vy * my + vz * mz) * r
            ov[sl] = 0.5 * acc
            return carry

        lax.fori_loop(0, ppw // _L, kappa_body, 0)
        pltpu.sync_copy(ov, kshared.at[bloc, pl.ds(base, ppw)])
        plsc.subcore_barrier()
        pltpu.sync_copy(kshared.at[bloc], kapv)

        def loss_body(i, acc):
            sl = pl.ds(i * _L, _L)
            gsl = pl.ds(base + i * _L, _L)
            sx, sy, sz = axv[gsl], ayv[gsl], azv[gsl]
            jn = i1v[sl]
            mx = plsc.load_gather(nxv, [jn])
            my = plsc.load_gather(nyv, [jn])
            mz = plsc.load_gather(nzv, [jn])
            ak = jnp.zeros((_L,), jnp.float32)
            for jv in (i21v[sl], i22v[sl]):
                vx = plsc.load_gather(axv, [jv]) - sx
                vy = plsc.load_gather(ayv, [jv]) - sy
                vz = plsc.load_gather(azv, [jv]) - sz
                r = _rsqrt(vx * vx + vy * vy + vz * vz)
                ak = ak + jnp.abs(vx * mx + vy * my + vz * mz) * r
            diff = 0.5 * ak - plsc.load_gather(kapv, [jn])
            return acc + diff * diff

        acc = lax.fori_loop(0, ppw // _L, loss_body,
                            jnp.zeros((_L,), jnp.float32))
        accv[...] = acc
        pltpu.sync_copy(accv, out_h.at[wid])

    return k(pcx, pcy, pcz, ax, ay, az, nx, ny, nz, io1, io2, i21, i22, i1n)


def kernel(pc_ori, input_curr_iter, normal_ori):
    b, _, n = pc_ori.shape
    pcx, pcy, pcz = pc_ori[:, 0], pc_ori[:, 1], pc_ori[:, 2]
    nx, ny, nz = normal_ori[:, 0], normal_ori[:, 1], normal_ori[:, 2]
    ax, ay, az = (input_curr_iter[:, 0], input_curr_iter[:, 1],
                  input_curr_iter[:, 2])
    io = _knn_self(pc_ori)
    i2s, i1s = _knn_adv(input_curr_iter, pc_ori)
    partials = _curv_sc(pcx, pcy, pcz, ax, ay, az, nx, ny, nz,
                        io[:, 0], io[:, 1], i2s[:, 0], i2s[:, 1], i1s[:, 0])
    return (10.0 / n) * jnp.sum(partials.reshape(b, -1), axis=1)


# speedup vs baseline: 1.0077x; 1.0077x over previous
import functools

import jax
import jax.numpy as jnp
from jax import lax
from jax.experimental import pallas as pl
from jax.experimental.pallas import tpu as pltpu
from jax.experimental.pallas import tpu_sc as plsc

_R = 1024
_L = 16


def _scores(q, db):
    cn = jnp.sum(db * db, axis=0, keepdims=True)
    return cn - 2.0 * lax.dot_general(
        q, db, (((0,), (0,)), ((), ())),
        preferred_element_type=jnp.float32)


def _select(s, colf, out_ref, nsel):
    inf = jnp.float32(jnp.inf)

    def argat(m):
        ii = jnp.min(jnp.where(s == m, colf, inf), axis=1, keepdims=True)
        return ii[:, 0].astype(jnp.int32)

    m1 = jnp.min(s, axis=1, keepdims=True)
    if nsel == 1:
        out_ref[0, 0, :] = argat(m1)
    else:
        m2 = jnp.min(jnp.where(s > m1, s, inf), axis=1, keepdims=True)
        out_ref[0, 0, :] = argat(m2)
        m3 = jnp.min(jnp.where(s > m2, s, inf), axis=1, keepdims=True)
        out_ref[0, 1, :] = argat(m3)


def _knn_self_body(q_ref, db_ref, out_ref):
    s = _scores(q_ref[0], db_ref[0])
    colf = lax.broadcasted_iota(jnp.int32, s.shape, 1).astype(jnp.float32)
    _select(s, colf, out_ref, 2)


def _knn_adv_body(q_ref, dba_ref, dbp_ref, outa_ref, outp_ref):
    sa = _scores(q_ref[0], dba_ref[0])
    colf = lax.broadcasted_iota(jnp.int32, sa.shape, 1).astype(jnp.float32)
    _select(sa, colf, outa_ref, 2)
    _select(_scores(q_ref[0], dbp_ref[0]), colf, outp_ref, 1)


def _knn_self(pc):
    b, _, n = pc.shape
    return pl.pallas_call(
        _knn_self_body,
        grid=(b, n // _R),
        in_specs=[
            pl.BlockSpec((1, 3, _R), lambda bb, r: (bb, 0, r)),
            pl.BlockSpec((1, 3, n), lambda bb, r: (bb, 0, 0)),
        ],
        out_specs=pl.BlockSpec((1, 2, _R), lambda bb, r: (bb, 0, r)),
        out_shape=jax.ShapeDtypeStruct((b, 2, n), jnp.int32),
    )(pc, pc)


def _knn_adv(adv, pc):
    b, _, n = adv.shape
    return pl.pallas_call(
        _knn_adv_body,
        grid=(b, n // _R),
        in_specs=[
            pl.BlockSpec((1, 3, _R), lambda bb, r: (bb, 0, r)),
            pl.BlockSpec((1, 3, n), lambda bb, r: (bb, 0, 0)),
            pl.BlockSpec((1, 3, n), lambda bb, r: (bb, 0, 0)),
        ],
        out_specs=[
            pl.BlockSpec((1, 2, _R), lambda bb, r: (bb, 0, r)),
            pl.BlockSpec((1, 1, _R), lambda bb, r: (bb, 0, r)),
        ],
        out_shape=[
            jax.ShapeDtypeStruct((b, 2, n), jnp.int32),
            jax.ShapeDtypeStruct((b, 1, n), jnp.int32),
        ],
    )(adv, adv, pc)


def _rsqrt(s):
    i = plsc.bitcast(s, jnp.int32)
    i = jnp.int32(0x5F3759DF) - (i >> 1)
    y = plsc.bitcast(i, jnp.float32)
    for _ in range(3):
        y = y * (1.5 - 0.5 * s * y * y)
    return y


def _sc_grid():
    info = plsc.get_sparse_core_info()
    return info.num_cores, info.num_subcores, info.num_cores * info.num_subcores


def _kappa_sc(pcx, pcy, pcz, nx, ny, nz, i1, i2):
    b, n = pcx.shape
    _, _, nw = _sc_grid()
    ppw = b * n // nw
    per_b = n // ppw
    mesh = plsc.VectorSubcoreMesh(core_axis_name="c", subcore_axis_name="s")

    @functools.partial(
        pl.kernel, mesh=mesh,
        compiler_params=pltpu.CompilerParams(needs_layout_passes=False),
        out_type=jax.ShapeDtypeStruct((b, n), jnp.float32),
        scratch_types=[
            pltpu.VMEM((n,), jnp.float32),
            pltpu.VMEM((n,), jnp.float32),
            pltpu.VMEM((n,), jnp.float32),
            pltpu.VMEM((ppw,), jnp.float32),
            pltpu.VMEM((ppw,), jnp.float32),
            pltpu.VMEM((ppw,), jnp.float32),
            pltpu.VMEM((ppw,), jnp.int32),
            pltpu.VMEM((ppw,), jnp.int32),
            pltpu.VMEM((ppw,), jnp.float32),
        ],
    )
    def k(pcx_h, pcy_h, pcz_h, nx_h, ny_h, nz_h, i1_h, i2_h, out_h,
          pxv, pyv, pzv, nxv, nyv, nzv, i1v, i2v, ov):
        nc, _, _ = _sc_grid()
        wid = lax.axis_index("s") * nc + lax.axis_index("c")
        bb = wid // per_b
        base = (wid % per_b) * ppw
        pltpu.sync_copy(pcx_h.at[bb], pxv)
        pltpu.sync_copy(pcy_h.at[bb], pyv)
        pltpu.sync_copy(pcz_h.at[bb], pzv)
        pltpu.sync_copy(nx_h.at[bb, pl.ds(base, ppw)], nxv)
        pltpu.sync_copy(ny_h.at[bb, pl.ds(base, ppw)], nyv)
        pltpu.sync_copy(nz_h.at[bb, pl.ds(base, ppw)], nzv)
        pltpu.sync_copy(i1_h.at[bb, pl.ds(base, ppw)], i1v)
        pltpu.sync_copy(i2_h.at[bb, pl.ds(base, ppw)], i2v)

        def body(i, carry):
            sl = pl.ds(i * _L, _L)
            gsl = pl.ds(base + i * _L, _L)
            sx, sy, sz = pxv[gsl], pyv[gsl], pzv[gsl]
            mx, my, mz = nxv[sl], nyv[sl], nzv[sl]
            acc = jnp.zeros((_L,), jnp.float32)
            for jv in (i1v[sl], i2v[sl]):
                vx = plsc.load_gather(pxv, [jv]) - sx
                vy = plsc.load_gather(pyv, [jv]) - sy
                vz = plsc.load_gather(pzv, [jv]) - sz
                r = _rsqrt(vx * vx + vy * vy + vz * vz)
                acc = acc + jnp.abs(vx * mx + vy * my + vz * mz) * r
            ov[sl] = 0.5 * acc
            return carry

        lax.fori_loop(0, ppw // _L, body, 0)
        pltpu.sync_copy(ov, out_h.at[bb, pl.ds(base, ppw)])

    return k(pcx, pcy, pcz, nx, ny, nz, i1, i2)


def _loss_sc(ax, ay, az, nx, ny, nz, kap, i21, i22, i1n):
    b, n = ax.shape
    _, _, nw = _sc_grid()
    ppw = b * n // nw
    per_b = n // ppw
    mesh = plsc.VectorSubcoreMesh(core_axis_name="c", subcore_axis_name="s")

    @functools.partial(
        pl.kernel, mesh=mesh,
        compiler_params=pltpu.CompilerParams(needs_layout_passes=False),
        out_type=jax.ShapeDtypeStruct((nw, _L), jnp.float32),
        scratch_types=[
            pltpu.VMEM((n,), jnp.float32),
            pltpu.VMEM((n,), jnp.float32),
            pltpu.VMEM((n,), jnp.float32),
            pltpu.VMEM((n,), jnp.float32),
            pltpu.VMEM((n,), jnp.float32),
            pltpu.VMEM((n,), jnp.float32),
            pltpu.VMEM((n,), jnp.float32),
            pltpu.VMEM((ppw,), jnp.int32),
            pltpu.VMEM((ppw,), jnp.int32),
            pltpu.VMEM((ppw,), jnp.int32),
            pltpu.VMEM((_L,), jnp.float32),
        ],
    )
    def k(ax_h, ay_h, az_h, nx_h, ny_h, nz_h, kap_h, i21_h, i22_h, i1_h,
          out_h, axv, ayv, azv, nxv, nyv, nzv, kapv, i21v, i22v, i1v, accv):
        nc, _, _ = _sc_grid()
        wid = lax.axis_index("s") * nc + lax.axis_index("c")
        bb = wid // per_b
        base = (wid % per_b) * ppw
        pltpu.sync_copy(ax_h.at[bb], axv)
        pltpu.sync_copy(ay_h.at[bb], ayv)
        pltpu.sync_copy(az_h.at[bb], azv)
        pltpu.sync_copy(nx_h.at[bb], nxv)
        pltpu.sync_copy(ny_h.at[bb], nyv)
        pltpu.sync_copy(nz_h.at[bb], nzv)
        pltpu.sync_copy(kap_h.at[bb], kapv)
        pltpu.sync_copy(i21_h.at[bb, pl.ds(base, ppw)], i21v)
        pltpu.sync_copy(i22_h.at[bb, pl.ds(base, ppw)], i22v)
        pltpu.sync_copy(i1_h.at[bb, pl.ds(base, ppw)], i1v)

        def body(i, acc):
            sl = pl.ds(i * _L, _L)
            gsl = pl.ds(base + i * _L, _L)
            sx, sy, sz = axv[gsl], ayv[gsl], azv[gsl]
            jn = i1v[sl]
            mx = plsc.load_gather(nxv, [jn])
            my = plsc.load_gather(nyv, [jn])
            mz = plsc.load_gather(nzv, [jn])
            ak = jnp.zeros((_L,), jnp.float32)
            for jv in (i21v[sl], i22v[sl]):
                vx = plsc.load_gather(axv, [jv]) - sx
                vy = plsc.load_gather(ayv, [jv]) - sy
                vz = plsc.load_gather(azv, [jv]) - sz
                r = _rsqrt(vx * vx + vy * vy + vz * vz)
                ak = ak + jnp.abs(vx * mx + vy * my + vz * mz) * r
            diff = 0.5 * ak - plsc.load_gather(kapv, [jn])
            return acc + diff * diff

        acc = lax.fori_loop(0, ppw // _L, body, jnp.zeros((_L,), jnp.float32))
        accv[...] = acc
        pltpu.sync_copy(accv, out_h.at[wid])

    return k(ax, ay, az, nx, ny, nz, kap, i21, i22, i1n)


def kernel(pc_ori, input_curr_iter, normal_ori):
    b, _, n = pc_ori.shape
    pcx, pcy, pcz = pc_ori[:, 0], pc_ori[:, 1], pc_ori[:, 2]
    nx, ny, nz = normal_ori[:, 0], normal_ori[:, 1], normal_ori[:, 2]
    ax, ay, az = (input_curr_iter[:, 0], input_curr_iter[:, 1],
                  input_curr_iter[:, 2])
    io = _knn_self(pc_ori)
    kap = _kappa_sc(pcx, pcy, pcz, nx, ny, nz, io[:, 0], io[:, 1])
    i2s, i1s = _knn_adv(input_curr_iter, pc_ori)
    i21, i22 = i2s[:, 0], i2s[:, 1]
    i1n = i1s[:, 0]
    partials = _loss_sc(ax, ay, az, nx, ny, nz, kap, i21, i22, i1n)
    return (10.0 / n) * jnp.sum(partials.reshape(b, -1), axis=1)
